# baseline (speedup 1.0000x reference)
import functools

import jax
import jax.numpy as jnp
from jax import lax
from jax.experimental import pallas as pl
from jax.experimental.pallas import tpu as pltpu
from jax.experimental.pallas import tpu_sc as plsc

N = 10000
E = 320000
C = 128
NC = 2
NS = 16
NW = NC * NS
NPAD = 10240
EPT = E // NW
DCH = 80
DNCH = EPT // DCH
PCH = 40
PNCH = EPT // PCH
PRING = 5
RPT = NPAD // NS

def _sc_deg_body(row_hbm, ea_hbm, out_hbm, rbig_v, wbig_v, ridx_v, wsm_v,
                 zb_v, acc_sh):
    cid = lax.axis_index("c")
    sid = lax.axis_index("s")
    wid = cid * NS + sid
    pltpu.sync_copy(row_hbm.at[pl.ds(wid * EPT, EPT)], rbig_v)
    pltpu.sync_copy(ea_hbm.at[pl.ds(wid * EPT, EPT)], wbig_v)

    def zero_zb(i, carry):
        zb_v[pl.ds(i * 16, 16)] = jnp.zeros((16,), jnp.float32)
        return carry

    lax.fori_loop(0, RPT // 16, zero_zb, 0)
    pltpu.sync_copy(zb_v, acc_sh.at[pl.ds(sid * RPT, RPT)])
    plsc.subcore_barrier()

    def chunk(i, carry):
        def cp(g, c2):
            ridx_v[pl.ds(g * 16, 16)] = rbig_v[pl.ds(i * DCH + g * 16, 16)]
            wsm_v[pl.ds(g * 16, 16)] = wbig_v[pl.ds(i * DCH + g * 16, 16)]
            return c2

        lax.fori_loop(0, DCH // 16, cp, 0)
        pltpu.sync_copy(wsm_v, acc_sh.at[ridx_v], add=True)
        return carry

    lax.fori_loop(0, DNCH, chunk, 0)
    plsc.subcore_barrier()
    pltpu.sync_copy(acc_sh.at[pl.ds(sid * RPT, RPT)],
                    out_hbm.at[pl.ds(cid * NPAD + sid * RPT, RPT)])


def _sc_prop_body(u_hbm, row_hbm, col_hbm, ea_hbm, out_hbm,
                  ridx_l, cidx_l, w_l, rows_l, acc_sh, msems, gsems, ssems):
    cid = lax.axis_index("c")
    sid = lax.axis_index("s")
    wid = cid * NS + sid
    base = wid * EPT

    def meta_descs(s, j):
        off = base + j * PCH
        return (
            pltpu.make_async_copy(row_hbm.at[pl.ds(off, PCH)],
                                  ridx_l[s], msems[s]),
            pltpu.make_async_copy(col_hbm.at[pl.ds(off, PCH)],
                                  cidx_l[s], msems[s]),
            pltpu.make_async_copy(ea_hbm.at[pl.ds(off, PCH)],
                                  w_l[s].at[pl.ds(0, PCH)], msems[s]),
        )

    def issue_meta(s, j):
        for d in meta_descs(s, j):
            d.start()

    def wait_meta(s, j):
        for d in meta_descs(s, j):
            d.wait()

    def g_desc(s):
        return pltpu.make_async_copy(u_hbm.at[ridx_l[s]], rows_l[s], gsems[s])

    def s_desc(s):
        return pltpu.make_async_copy(rows_l[s], acc_sh.at[cidx_l[s]],
                                     ssems[s])

    def zero_rows(i, carry):
        for c8 in range(C // 16):
            rows_l[0][i, pl.ds(c8 * 16, 16)] = jnp.zeros((16,), jnp.float32)
        return carry

    lax.fori_loop(0, PCH, zero_rows, 0)
    for r in range(RPT // PCH):
        pltpu.sync_copy(rows_l[0], acc_sh.at[pl.ds(sid * RPT + r * PCH, PCH)])
    plsc.subcore_barrier()

    issue_meta(0, 0)
    issue_meta(1, 1)
    wait_meta(0, 0)
    g_desc(0).start()

    def super_chunk(t, carry):
        for s in range(PRING):
            j = t * PRING + s
            s1 = (s + 1) % PRING
            s2 = (s + 2) % PRING

            @pl.when(j >= 3)
            def _():
                s_desc(s2).wait()

            @pl.when(j + 2 < PNCH)
            def _():
                issue_meta(s2, j + 2)

            @pl.when(j + 1 < PNCH)
            def _():
                wait_meta(s1, j + 1)
                g_desc(s1).start()

            g_desc(s).wait()

            def grp(g, c2, _s=s):
                w16 = w_l[_s][pl.ds(g * 8, 16)]
                for l in range(8):
                    wl = w16[l]
                    e = g * 8 + l
                    for c8 in range(C // 16):
                        rows_l[_s][e, pl.ds(c8 * 16, 16)] = (
                            rows_l[_s][e, pl.ds(c8 * 16, 16)] * wl)
                return c2

            lax.fori_loop(0, PCH // 8, grp, 0)
            s_desc(s).start(add=True)
        return carry

    lax.fori_loop(0, PNCH // PRING, super_chunk, 0)
    for s in (2, 3, 4):
        s_desc(s).wait()
    plsc.subcore_barrier()
    for r in range(RPT // PCH):
        pltpu.sync_copy(acc_sh.at[pl.ds(sid * RPT + r * PCH, PCH)],
                        out_hbm.at[cid].at[pl.ds(sid * RPT + r * PCH, PCH)])


@functools.cache
def _sc_kernels():
    mesh = plsc.VectorSubcoreMesh(
        core_axis_name="c", subcore_axis_name="s",
        num_cores=NC, num_subcores=NS)
    sc_deg = pl.kernel(
        _sc_deg_body,
        out_type=jax.ShapeDtypeStruct((NC * NPAD,), jnp.float32),
        mesh=mesh,
        scratch_types=[
            pltpu.VMEM((EPT,), jnp.int32),
            pltpu.VMEM((EPT,), jnp.float32),
            pltpu.VMEM((DCH,), jnp.int32),
            pltpu.VMEM((DCH,), jnp.float32),
            pltpu.VMEM((RPT,), jnp.float32),
            pltpu.VMEM_SHARED((NPAD,), jnp.float32),
        ],
    )
    sc_prop = pl.kernel(
        _sc_prop_body,
        out_type=jax.ShapeDtypeStruct((NC, NPAD, C), jnp.float32),
        mesh=mesh,
        scratch_types=[
            tuple(pltpu.VMEM((PCH,), jnp.int32) for _ in range(PRING)),
            tuple(pltpu.VMEM((PCH,), jnp.int32) for _ in range(PRING)),
            tuple(pltpu.VMEM((48,), jnp.float32) for _ in range(PRING)),
            tuple(pltpu.VMEM((PCH, C), jnp.float32) for _ in range(PRING)),
            pltpu.VMEM_SHARED((NPAD, C), jnp.float32),
            tuple(pltpu.SemaphoreType.DMA for _ in range(PRING)),
            tuple(pltpu.SemaphoreType.DMA for _ in range(PRING)),
            tuple(pltpu.SemaphoreType.DMA for _ in range(PRING)),
        ],
    )
    return sc_deg, sc_prop


BS = 1000
GRID = N // BS


def _tc_pre_body(dp_ref, x_ref, dinv_ref, u0_ref):
    deg = dp_ref[0] + dp_ref[1]
    pos = deg > 0.0
    dinv = jnp.where(pos, lax.rsqrt(jnp.where(pos, deg, 1.0)), 0.0)
    dinv_ref[...] = dinv
    u0_ref[...] = dinv * x_ref[...]


_tc_pre = pl.pallas_call(
    _tc_pre_body,
    grid=(GRID,),
    in_specs=[
        pl.BlockSpec((NC, BS, 1), lambda i: (0, i, 0)),
        pl.BlockSpec((BS, C), lambda i: (i, 0)),
    ],
    out_specs=[
        pl.BlockSpec((BS, 1), lambda i: (i, 0)),
        pl.BlockSpec((BS, C), lambda i: (i, 0)),
    ],
    out_shape=[
        jax.ShapeDtypeStruct((N, 1), jnp.float32),
        jax.ShapeDtypeStruct((N, C), jnp.float32),
    ],
)


def _tc_mid_body(yp_ref, dinv_ref, x_ref, w0_ref, w1_ref, u1_ref, part_ref):
    y = yp_ref[0] + yp_ref[1]
    dinv = dinv_ref[...]
    tx1 = -dinv * y
    u1_ref[...] = dinv * tx1
    part_ref[...] = (
        jnp.dot(x_ref[...], w0_ref[...], preferred_element_type=jnp.float32)
        + jnp.dot(tx1, w1_ref[...], preferred_element_type=jnp.float32))


_tc_mid = pl.pallas_call(
    _tc_mid_body,
    grid=(GRID,),
    in_specs=[
        pl.BlockSpec((NC, BS, C), lambda i: (0, i, 0)),
        pl.BlockSpec((BS, 1), lambda i: (i, 0)),
        pl.BlockSpec((BS, C), lambda i: (i, 0)),
        pl.BlockSpec((C, C), lambda i: (0, 0)),
        pl.BlockSpec((C, C), lambda i: (0, 0)),
    ],
    out_specs=[
        pl.BlockSpec((BS, C), lambda i: (i, 0)),
        pl.BlockSpec((BS, C), lambda i: (i, 0)),
    ],
    out_shape=[
        jax.ShapeDtypeStruct((N, C), jnp.float32),
        jax.ShapeDtypeStruct((N, C), jnp.float32),
    ],
)


def _tc_post_body(yp_ref, dinv_ref, x_ref, part_ref, w2_ref, b_ref, o_ref):
    y = yp_ref[0] + yp_ref[1]
    tx2 = -2.0 * dinv_ref[...] * y - x_ref[...]
    o = (part_ref[...]
         + jnp.dot(tx2, w2_ref[...], preferred_element_type=jnp.float32)
         + b_ref[...])
    o_ref[...] = jnp.where(o >= 0.0, o, 0.01 * o)


_tc_post = pl.pallas_call(
    _tc_post_body,
    grid=(GRID,),
    in_specs=[
        pl.BlockSpec((NC, BS, C), lambda i: (0, i, 0)),
        pl.BlockSpec((BS, 1), lambda i: (i, 0)),
        pl.BlockSpec((BS, C), lambda i: (i, 0)),
        pl.BlockSpec((BS, C), lambda i: (i, 0)),
        pl.BlockSpec((C, C), lambda i: (0, 0)),
        pl.BlockSpec((1, C), lambda i: (0, 0)),
    ],
    out_specs=pl.BlockSpec((BS, C), lambda i: (i, 0)),
    out_shape=jax.ShapeDtypeStruct((N, C), jnp.float32),
)


def kernel(x, edgeIndex, edgeAttribute, W, b):
    row = edgeIndex[0]
    col = edgeIndex[1]
    _sc_deg, _sc_prop = _sc_kernels()

    deg_part = _sc_deg(row, edgeAttribute)
    dp = deg_part.reshape(NC, NPAD, 1)
    dinv, u0 = _tc_pre(dp, x)
    y1p = _sc_prop(u0, row, col, edgeAttribute)
    u1, part = _tc_mid(y1p, dinv, x, W[0], W[1])
    y2p = _sc_prop(u1, row, col, edgeAttribute)
    out = _tc_post(y2p, dinv, x, part, W[2], b.reshape(1, C))
    return out


# speedup vs baseline: 1.0175x; 1.0175x over previous
import functools

import jax
import jax.numpy as jnp
from jax import lax
from jax.experimental import pallas as pl
from jax.experimental.pallas import tpu as pltpu
from jax.experimental.pallas import tpu_sc as plsc

N = 10000
E = 320000
C = 128
NC = 2
NS = 16
NW = NC * NS
NPAD = 10240
EPT = E // NW
DCH = 80
DNCH = EPT // DCH
PCH = 80
PNCH = EPT // PCH
PRING = 4
RPT = NPAD // NS

def _sc_deg_body(row_hbm, ea_hbm, out_hbm, rbig_v, wbig_v, ridx_v, wsm_v,
                 zb_v, acc_sh):
    cid = lax.axis_index("c")
    sid = lax.axis_index("s")
    wid = cid * NS + sid
    pltpu.sync_copy(row_hbm.at[pl.ds(wid * EPT, EPT)], rbig_v)
    pltpu.sync_copy(ea_hbm.at[pl.ds(wid * EPT, EPT)], wbig_v)

    def zero_zb(i, carry):
        zb_v[pl.ds(i * 16, 16)] = jnp.zeros((16,), jnp.float32)
        return carry

    lax.fori_loop(0, RPT // 16, zero_zb, 0)
    pltpu.sync_copy(zb_v, acc_sh.at[pl.ds(sid * RPT, RPT)])
    plsc.subcore_barrier()

    def chunk(i, carry):
        def cp(g, c2):
            ridx_v[pl.ds(g * 16, 16)] = rbig_v[pl.ds(i * DCH + g * 16, 16)]
            wsm_v[pl.ds(g * 16, 16)] = wbig_v[pl.ds(i * DCH + g * 16, 16)]
            return c2

        lax.fori_loop(0, DCH // 16, cp, 0)
        pltpu.sync_copy(wsm_v, acc_sh.at[ridx_v], add=True)
        return carry

    lax.fori_loop(0, DNCH, chunk, 0)
    plsc.subcore_barrier()
    pltpu.sync_copy(acc_sh.at[pl.ds(sid * RPT, RPT)],
                    out_hbm.at[pl.ds(cid * NPAD + sid * RPT, RPT)])


def _sc_prop_body(u_hbm, row_hbm, col_hbm, ea_hbm, out_hbm,
                  ridx_l, cidx_l, w_l, rows_l, acc_sh, msems, gsems, ssems):
    cid = lax.axis_index("c")
    sid = lax.axis_index("s")
    wid = cid * NS + sid
    base = wid * EPT

    def meta_descs(s, j):
        off = base + j * PCH
        return (
            pltpu.make_async_copy(row_hbm.at[pl.ds(off, PCH)],
                                  ridx_l[s], msems[s]),
            pltpu.make_async_copy(col_hbm.at[pl.ds(off, PCH)],
                                  cidx_l[s], msems[s]),
            pltpu.make_async_copy(ea_hbm.at[pl.ds(off, PCH)],
                                  w_l[s].at[pl.ds(0, PCH)], msems[s]),
        )

    def issue_meta(s, j):
        for d in meta_descs(s, j):
            d.start()

    def wait_meta(s, j):
        for d in meta_descs(s, j):
            d.wait()

    def g_desc(s):
        return pltpu.make_async_copy(u_hbm.at[ridx_l[s]], rows_l[s], gsems[s])

    def s_desc(s):
        return pltpu.make_async_copy(rows_l[s], acc_sh.at[cidx_l[s]],
                                     ssems[s])

    def zero_rows(i, carry):
        for c8 in range(C // 16):
            rows_l[0][i, pl.ds(c8 * 16, 16)] = jnp.zeros((16,), jnp.float32)
        return carry

    lax.fori_loop(0, PCH, zero_rows, 0)
    for r in range(RPT // PCH):
        pltpu.sync_copy(rows_l[0], acc_sh.at[pl.ds(sid * RPT + r * PCH, PCH)])
    plsc.subcore_barrier()

    issue_meta(0, 0)
    issue_meta(1, 1)
    wait_meta(0, 0)
    g_desc(0).start()

    def scale(s):
        def grp(g, c2, _s=s):
            w16 = w_l[_s][pl.ds(g * 8, 16)]
            for l in range(8):
                wl = w16[l]
                e = g * 8 + l
                for c8 in range(C // 16):
                    rows_l[_s][e, pl.ds(c8 * 16, 16)] = (
                        rows_l[_s][e, pl.ds(c8 * 16, 16)] * wl)
            return c2

        lax.fori_loop(0, PCH // 8, grp, 0)

    def super_chunk(t, carry):
        for s in range(PRING):
            j = t * PRING + s
            s1 = (s + 1) % PRING
            s2 = (s + 2) % PRING

            @pl.when(j >= 2)
            def _():
                s_desc(s2).wait()

            @pl.when(j + 2 < PNCH)
            def _():
                issue_meta(s2, j + 2)

            @pl.when(j + 1 < PNCH)
            def _():
                wait_meta(s1, j + 1)
                g_desc(s1).start()

            g_desc(s).wait()
            scale(s)
            s_desc(s).start(add=True)
        return carry

    lax.fori_loop(0, PNCH // PRING, super_chunk, 0)
    g_desc(0).wait()
    scale(0)
    s_desc(0).start(add=True)
    for s in (2, 3, 0):
        s_desc(s).wait()
    plsc.subcore_barrier()
    for r in range(RPT // PCH):
        pltpu.sync_copy(acc_sh.at[pl.ds(sid * RPT + r * PCH, PCH)],
                        out_hbm.at[cid].at[pl.ds(sid * RPT + r * PCH, PCH)])


@functools.cache
def _sc_kernels():
    mesh = plsc.VectorSubcoreMesh(
        core_axis_name="c", subcore_axis_name="s",
        num_cores=NC, num_subcores=NS)
    sc_deg = pl.kernel(
        _sc_deg_body,
        out_type=jax.ShapeDtypeStruct((NC * NPAD,), jnp.float32),
        mesh=mesh,
        scratch_types=[
            pltpu.VMEM((EPT,), jnp.int32),
            pltpu.VMEM((EPT,), jnp.float32),
            pltpu.VMEM((DCH,), jnp.int32),
            pltpu.VMEM((DCH,), jnp.float32),
            pltpu.VMEM((RPT,), jnp.float32),
            pltpu.VMEM_SHARED((NPAD,), jnp.float32),
        ],
    )
    sc_prop = pl.kernel(
        _sc_prop_body,
        out_type=jax.ShapeDtypeStruct((NC, NPAD, C), jnp.float32),
        mesh=mesh,
        scratch_types=[
            tuple(pltpu.VMEM((PCH,), jnp.int32) for _ in range(PRING)),
            tuple(pltpu.VMEM((PCH,), jnp.int32) for _ in range(PRING)),
            tuple(pltpu.VMEM((96,), jnp.float32) for _ in range(PRING)),
            tuple(pltpu.VMEM((PCH, C), jnp.float32) for _ in range(PRING)),
            pltpu.VMEM_SHARED((NPAD, C), jnp.float32),
            tuple(pltpu.SemaphoreType.DMA for _ in range(PRING)),
            tuple(pltpu.SemaphoreType.DMA for _ in range(PRING)),
            tuple(pltpu.SemaphoreType.DMA for _ in range(PRING)),
        ],
    )
    return sc_deg, sc_prop


BS = 1000
GRID = N // BS


def _tc_pre_body(dp_ref, x_ref, dinv_ref, u0_ref):
    deg = dp_ref[0] + dp_ref[1]
    pos = deg > 0.0
    dinv = jnp.where(pos, lax.rsqrt(jnp.where(pos, deg, 1.0)), 0.0)
    dinv_ref[...] = dinv
    u0_ref[...] = dinv * x_ref[...]


_tc_pre = pl.pallas_call(
    _tc_pre_body,
    grid=(GRID,),
    in_specs=[
        pl.BlockSpec((NC, BS, 1), lambda i: (0, i, 0)),
        pl.BlockSpec((BS, C), lambda i: (i, 0)),
    ],
    out_specs=[
        pl.BlockSpec((BS, 1), lambda i: (i, 0)),
        pl.BlockSpec((BS, C), lambda i: (i, 0)),
    ],
    out_shape=[
        jax.ShapeDtypeStruct((N, 1), jnp.float32),
        jax.ShapeDtypeStruct((N, C), jnp.float32),
    ],
)


def _tc_mid_body(yp_ref, dinv_ref, x_ref, w0_ref, w1_ref, u1_ref, part_ref):
    y = yp_ref[0] + yp_ref[1]
    dinv = dinv_ref[...]
    tx1 = -dinv * y
    u1_ref[...] = dinv * tx1
    part_ref[...] = (
        jnp.dot(x_ref[...], w0_ref[...], preferred_element_type=jnp.float32)
        + jnp.dot(tx1, w1_ref[...], preferred_element_type=jnp.float32))


_tc_mid = pl.pallas_call(
    _tc_mid_body,
    grid=(GRID,),
    in_specs=[
        pl.BlockSpec((NC, BS, C), lambda i: (0, i, 0)),
        pl.BlockSpec((BS, 1), lambda i: (i, 0)),
        pl.BlockSpec((BS, C), lambda i: (i, 0)),
        pl.BlockSpec((C, C), lambda i: (0, 0)),
        pl.BlockSpec((C, C), lambda i: (0, 0)),
    ],
    out_specs=[
        pl.BlockSpec((BS, C), lambda i: (i, 0)),
        pl.BlockSpec((BS, C), lambda i: (i, 0)),
    ],
    out_shape=[
        jax.ShapeDtypeStruct((N, C), jnp.float32),
        jax.ShapeDtypeStruct((N, C), jnp.float32),
    ],
)


def _tc_post_body(yp_ref, dinv_ref, x_ref, part_ref, w2_ref, b_ref, o_ref):
    y = yp_ref[0] + yp_ref[1]
    tx2 = -2.0 * dinv_ref[...] * y - x_ref[...]
    o = (part_ref[...]
         + jnp.dot(tx2, w2_ref[...], preferred_element_type=jnp.float32)
         + b_ref[...])
    o_ref[...] = jnp.where(o >= 0.0, o, 0.01 * o)


_tc_post = pl.pallas_call(
    _tc_post_body,
    grid=(GRID,),
    in_specs=[
        pl.BlockSpec((NC, BS, C), lambda i: (0, i, 0)),
        pl.BlockSpec((BS, 1), lambda i: (i, 0)),
        pl.BlockSpec((BS, C), lambda i: (i, 0)),
        pl.BlockSpec((BS, C), lambda i: (i, 0)),
        pl.BlockSpec((C, C), lambda i: (0, 0)),
        pl.BlockSpec((1, C), lambda i: (0, 0)),
    ],
    out_specs=pl.BlockSpec((BS, C), lambda i: (i, 0)),
    out_shape=jax.ShapeDtypeStruct((N, C), jnp.float32),
)


def kernel(x, edgeIndex, edgeAttribute, W, b):
    row = edgeIndex[0]
    col = edgeIndex[1]
    _sc_deg, _sc_prop = _sc_kernels()

    deg_part = _sc_deg(row, edgeAttribute)
    dp = deg_part.reshape(NC, NPAD, 1)
    dinv, u0 = _tc_pre(dp, x)
    y1p = _sc_prop(u0, row, col, edgeAttribute)
    u1, part = _tc_mid(y1p, dinv, x, W[0], W[1])
    y2p = _sc_prop(u1, row, col, edgeAttribute)
    out = _tc_post(y2p, dinv, x, part, W[2], b.reshape(1, C))
    return out


# speedup vs baseline: 2.3254x; 2.2854x over previous
import functools

import jax
import jax.numpy as jnp
from jax import lax
from jax.experimental import pallas as pl
from jax.experimental.pallas import tpu as pltpu
from jax.experimental.pallas import tpu_sc as plsc

N = 10000
E = 320000
C = 128
NC = 2
NS = 16
NW = NC * NS
NPAD = 10240
EPT = E // NW
DCH = 80
DNCH = EPT // DCH
PCH = 80
PNCH = EPT // PCH
MRING = 4
DRING = 2
RPT = NPAD // NS

def _sc_deg_body(row_hbm, ea_hbm, out_hbm, rbig_v, wbig_v, ridx_v, wsm_v,
                 zb_v, acc_sh):
    cid = lax.axis_index("c")
    sid = lax.axis_index("s")
    wid = cid * NS + sid
    pltpu.sync_copy(row_hbm.at[pl.ds(wid * EPT, EPT)], rbig_v)
    pltpu.sync_copy(ea_hbm.at[pl.ds(wid * EPT, EPT)], wbig_v)

    def zero_zb(i, carry):
        zb_v[pl.ds(i * 16, 16)] = jnp.zeros((16,), jnp.float32)
        return carry

    lax.fori_loop(0, RPT // 16, zero_zb, 0)
    pltpu.sync_copy(zb_v, acc_sh.at[pl.ds(sid * RPT, RPT)])
    plsc.subcore_barrier()

    def chunk(i, carry):
        def cp(g, c2):
            ridx_v[pl.ds(g * 16, 16)] = rbig_v[pl.ds(i * DCH + g * 16, 16)]
            wsm_v[pl.ds(g * 16, 16)] = wbig_v[pl.ds(i * DCH + g * 16, 16)]
            return c2

        lax.fori_loop(0, DCH // 16, cp, 0)
        pltpu.sync_copy(wsm_v, acc_sh.at[ridx_v], add=True)
        return carry

    lax.fori_loop(0, DNCH, chunk, 0)
    plsc.subcore_barrier()
    pltpu.sync_copy(acc_sh.at[pl.ds(sid * RPT, RPT)],
                    out_hbm.at[pl.ds(cid * NPAD + sid * RPT, RPT)])


def _sc_prop_body(u_hbm, row_hbm, col_hbm, ea_hbm, out_hbm,
                  ridx_l, cidx_l, w_l, rows_l, msg_l, acc_sh,
                  msems, gsems, ssems):
    cid = lax.axis_index("c")
    sid = lax.axis_index("s")
    wid = cid * NS + sid
    base = wid * EPT

    def meta_descs(s4, j):
        off = base + j * PCH
        return (
            pltpu.make_async_copy(row_hbm.at[pl.ds(off, PCH)],
                                  ridx_l[s4], msems[s4]),
            pltpu.make_async_copy(col_hbm.at[pl.ds(off, PCH)],
                                  cidx_l[s4], msems[s4]),
            pltpu.make_async_copy(ea_hbm.at[pl.ds(off, PCH)],
                                  w_l[s4].at[pl.ds(0, PCH)], msems[s4]),
        )

    def issue_meta(s4, j):
        for d in meta_descs(s4, j):
            d.start()

    def wait_meta(s4, j):
        for d in meta_descs(s4, j):
            d.wait()

    def g_desc(s2, s4):
        return pltpu.make_async_copy(u_hbm.at[ridx_l[s4]], rows_l[s2],
                                     gsems[s2])

    def s_desc(s2, s4):
        return pltpu.make_async_copy(msg_l[s2], acc_sh.at[cidx_l[s4]],
                                     ssems[s2])

    def zero_rows(i, carry):
        for c8 in range(C // 16):
            msg_l[0][i, pl.ds(c8 * 16, 16)] = jnp.zeros((16,), jnp.float32)
        return carry

    lax.fori_loop(0, PCH, zero_rows, 0)
    for r in range(RPT // PCH):
        pltpu.sync_copy(msg_l[0], acc_sh.at[pl.ds(sid * RPT + r * PCH, PCH)])
    plsc.subcore_barrier()

    issue_meta(0, 0)
    issue_meta(1, 1)
    wait_meta(0, 0)
    g_desc(0, 0).start()

    def scale(s2, s4):
        @plsc.parallel_loop(0, PCH, step=8, unroll=2)
        def _(e0):
            w16 = w_l[s4][pl.ds(e0, 16)]
            for l in range(8):
                wl = w16[l]
                for c8 in range(C // 16):
                    msg_l[s2][e0 + l, pl.ds(c8 * 16, 16)] = (
                        rows_l[s2][e0 + l, pl.ds(c8 * 16, 16)] * wl)

    def super_chunk(t, carry):
        for k in range(MRING):
            j = t * MRING + k
            s2 = k % DRING
            s4 = k
            s2n = (k + 1) % DRING
            s4n = (k + 1) % MRING
            s4p = (k + 2) % MRING

            @pl.when(j >= 2)
            def _():
                s_desc(s2, s4p).wait()

            @pl.when(j + 2 < PNCH)
            def _():
                issue_meta(s4p, j + 2)

            @pl.when(j + 1 < PNCH)
            def _():
                wait_meta(s4n, j + 1)
                g_desc(s2n, s4n).start()

            g_desc(s2, s4).wait()
            scale(s2, s4)
            s_desc(s2, s4).start(add=True)
        return carry

    lax.fori_loop(0, PNCH // MRING, super_chunk, 0)
    s_desc(0, 2).wait()
    g_desc(0, 0).wait()
    scale(0, 0)
    s_desc(0, 0).start(add=True)
    s_desc(1, 3).wait()
    s_desc(0, 0).wait()
    plsc.subcore_barrier()
    for r in range(RPT // PCH):
        pltpu.sync_copy(acc_sh.at[pl.ds(sid * RPT + r * PCH, PCH)],
                        out_hbm.at[cid].at[pl.ds(sid * RPT + r * PCH, PCH)])


@functools.cache
def _sc_kernels():
    mesh = plsc.VectorSubcoreMesh(
        core_axis_name="c", subcore_axis_name="s",
        num_cores=NC, num_subcores=NS)
    sc_deg = pl.kernel(
        _sc_deg_body,
        out_type=jax.ShapeDtypeStruct((NC * NPAD,), jnp.float32),
        mesh=mesh,
        scratch_types=[
            pltpu.VMEM((EPT,), jnp.int32),
            pltpu.VMEM((EPT,), jnp.float32),
            pltpu.VMEM((DCH,), jnp.int32),
            pltpu.VMEM((DCH,), jnp.float32),
            pltpu.VMEM((RPT,), jnp.float32),
            pltpu.VMEM_SHARED((NPAD,), jnp.float32),
        ],
    )
    sc_prop = pl.kernel(
        _sc_prop_body,
        out_type=jax.ShapeDtypeStruct((NC, NPAD, C), jnp.float32),
        mesh=mesh,
        scratch_types=[
            tuple(pltpu.VMEM((PCH,), jnp.int32) for _ in range(MRING)),
            tuple(pltpu.VMEM((PCH,), jnp.int32) for _ in range(MRING)),
            tuple(pltpu.VMEM((96,), jnp.float32) for _ in range(MRING)),
            tuple(pltpu.VMEM((PCH, C), jnp.float32) for _ in range(DRING)),
            tuple(pltpu.VMEM((PCH, C), jnp.float32) for _ in range(DRING)),
            pltpu.VMEM_SHARED((NPAD, C), jnp.float32),
            tuple(pltpu.SemaphoreType.DMA for _ in range(MRING)),
            tuple(pltpu.SemaphoreType.DMA for _ in range(DRING)),
            tuple(pltpu.SemaphoreType.DMA for _ in range(DRING)),
        ],
    )
    return sc_deg, sc_prop


BS = 1000
GRID = N // BS


def _tc_pre_body(dp_ref, x_ref, dinv_ref, u0_ref):
    deg = dp_ref[0] + dp_ref[1]
    pos = deg > 0.0
    dinv = jnp.where(pos, lax.rsqrt(jnp.where(pos, deg, 1.0)), 0.0)
    dinv_ref[...] = dinv
    u0_ref[...] = dinv * x_ref[...]


_tc_pre = pl.pallas_call(
    _tc_pre_body,
    grid=(GRID,),
    in_specs=[
        pl.BlockSpec((NC, BS, 1), lambda i: (0, i, 0)),
        pl.BlockSpec((BS, C), lambda i: (i, 0)),
    ],
    out_specs=[
        pl.BlockSpec((BS, 1), lambda i: (i, 0)),
        pl.BlockSpec((BS, C), lambda i: (i, 0)),
    ],
    out_shape=[
        jax.ShapeDtypeStruct((N, 1), jnp.float32),
        jax.ShapeDtypeStruct((N, C), jnp.float32),
    ],
)


def _tc_mid_body(yp_ref, dinv_ref, x_ref, w0_ref, w1_ref, u1_ref, part_ref):
    y = yp_ref[0] + yp_ref[1]
    dinv = dinv_ref[...]
    tx1 = -dinv * y
    u1_ref[...] = dinv * tx1
    part_ref[...] = (
        jnp.dot(x_ref[...], w0_ref[...], preferred_element_type=jnp.float32)
        + jnp.dot(tx1, w1_ref[...], preferred_element_type=jnp.float32))


_tc_mid = pl.pallas_call(
    _tc_mid_body,
    grid=(GRID,),
    in_specs=[
        pl.BlockSpec((NC, BS, C), lambda i: (0, i, 0)),
        pl.BlockSpec((BS, 1), lambda i: (i, 0)),
        pl.BlockSpec((BS, C), lambda i: (i, 0)),
        pl.BlockSpec((C, C), lambda i: (0, 0)),
        pl.BlockSpec((C, C), lambda i: (0, 0)),
    ],
    out_specs=[
        pl.BlockSpec((BS, C), lambda i: (i, 0)),
        pl.BlockSpec((BS, C), lambda i: (i, 0)),
    ],
    out_shape=[
        jax.ShapeDtypeStruct((N, C), jnp.float32),
        jax.ShapeDtypeStruct((N, C), jnp.float32),
    ],
)


def _tc_post_body(yp_ref, dinv_ref, x_ref, part_ref, w2_ref, b_ref, o_ref):
    y = yp_ref[0] + yp_ref[1]
    tx2 = -2.0 * dinv_ref[...] * y - x_ref[...]
    o = (part_ref[...]
         + jnp.dot(tx2, w2_ref[...], preferred_element_type=jnp.float32)
         + b_ref[...])
    o_ref[...] = jnp.where(o >= 0.0, o, 0.01 * o)


_tc_post = pl.pallas_call(
    _tc_post_body,
    grid=(GRID,),
    in_specs=[
        pl.BlockSpec((NC, BS, C), lambda i: (0, i, 0)),
        pl.BlockSpec((BS, 1), lambda i: (i, 0)),
        pl.BlockSpec((BS, C), lambda i: (i, 0)),
        pl.BlockSpec((BS, C), lambda i: (i, 0)),
        pl.BlockSpec((C, C), lambda i: (0, 0)),
        pl.BlockSpec((1, C), lambda i: (0, 0)),
    ],
    out_specs=pl.BlockSpec((BS, C), lambda i: (i, 0)),
    out_shape=jax.ShapeDtypeStruct((N, C), jnp.float32),
)


def kernel(x, edgeIndex, edgeAttribute, W, b):
    row = edgeIndex[0]
    col = edgeIndex[1]
    _sc_deg, _sc_prop = _sc_kernels()

    deg_part = _sc_deg(row, edgeAttribute)
    dp = deg_part.reshape(NC, NPAD, 1)
    dinv, u0 = _tc_pre(dp, x)
    y1p = _sc_prop(u0, row, col, edgeAttribute)
    u1, part = _tc_mid(y1p, dinv, x, W[0], W[1])
    y2p = _sc_prop(u1, row, col, edgeAttribute)
    out = _tc_post(y2p, dinv, x, part, W[2], b.reshape(1, C))
    return out


# speedup vs baseline: 2.3793x; 1.0232x over previous
import functools

import jax
import jax.numpy as jnp
from jax import lax
from jax.experimental import pallas as pl
from jax.experimental.pallas import tpu as pltpu
from jax.experimental.pallas import tpu_sc as plsc

N = 10000
E = 320000
C = 128
NC = 2
NS = 16
NW = NC * NS
NPAD = 10240
EPT = E // NW
DCH = 80
DNCH = EPT // DCH
PCH = 80
PNCH = EPT // PCH
MRING = 8
DRING = 4
SUPER = 8
NTAIL = PNCH % SUPER
RPT = NPAD // NS

def _sc_deg_body(row_hbm, ea_hbm, out_hbm, rbig_v, wbig_v, ridx_v, wsm_v,
                 zb_v, acc_sh):
    cid = lax.axis_index("c")
    sid = lax.axis_index("s")
    wid = cid * NS + sid
    pltpu.sync_copy(row_hbm.at[pl.ds(wid * EPT, EPT)], rbig_v)
    pltpu.sync_copy(ea_hbm.at[pl.ds(wid * EPT, EPT)], wbig_v)

    def zero_zb(i, carry):
        zb_v[pl.ds(i * 16, 16)] = jnp.zeros((16,), jnp.float32)
        return carry

    lax.fori_loop(0, RPT // 16, zero_zb, 0)
    pltpu.sync_copy(zb_v, acc_sh.at[pl.ds(sid * RPT, RPT)])
    plsc.subcore_barrier()

    def chunk(i, carry):
        def cp(g, c2):
            ridx_v[pl.ds(g * 16, 16)] = rbig_v[pl.ds(i * DCH + g * 16, 16)]
            wsm_v[pl.ds(g * 16, 16)] = wbig_v[pl.ds(i * DCH + g * 16, 16)]
            return c2

        lax.fori_loop(0, DCH // 16, cp, 0)
        pltpu.sync_copy(wsm_v, acc_sh.at[ridx_v], add=True)
        return carry

    lax.fori_loop(0, DNCH, chunk, 0)
    plsc.subcore_barrier()
    pltpu.sync_copy(acc_sh.at[pl.ds(sid * RPT, RPT)],
                    out_hbm.at[pl.ds(cid * NPAD + sid * RPT, RPT)])


def _sc_prop_body(u_hbm, row_hbm, col_hbm, ea_hbm, out_hbm,
                  ridx_l, cidx_l, w_l, rows_l, acc_sh,
                  msems, gsems, ssems):
    cid = lax.axis_index("c")
    sid = lax.axis_index("s")
    wid = cid * NS + sid
    base = wid * EPT

    def meta_descs(sm, j):
        off = base + j * PCH
        return (
            pltpu.make_async_copy(row_hbm.at[pl.ds(off, PCH)],
                                  ridx_l[sm], msems[sm]),
            pltpu.make_async_copy(col_hbm.at[pl.ds(off, PCH)],
                                  cidx_l[sm], msems[sm]),
            pltpu.make_async_copy(ea_hbm.at[pl.ds(off, PCH)],
                                  w_l[sm].at[pl.ds(0, PCH)], msems[sm]),
        )

    def issue_meta(sm, j):
        for d in meta_descs(sm, j):
            d.start()

    def wait_meta(sm, j):
        for d in meta_descs(sm, j):
            d.wait()

    def g_desc(sr, sm):
        return pltpu.make_async_copy(u_hbm.at[ridx_l[sm]], rows_l[sr],
                                     gsems[sr])

    def s_desc(sr, sm):
        return pltpu.make_async_copy(rows_l[sr], acc_sh.at[cidx_l[sm]],
                                     ssems[sr])

    def zero_rows(i, carry):
        for c8 in range(C // 16):
            rows_l[0][i, pl.ds(c8 * 16, 16)] = jnp.zeros((16,), jnp.float32)
        return carry

    lax.fori_loop(0, PCH, zero_rows, 0)
    for r in range(RPT // PCH):
        pltpu.sync_copy(rows_l[0], acc_sh.at[pl.ds(sid * RPT + r * PCH, PCH)])
    plsc.subcore_barrier()

    def scale(sr, sm):
        @plsc.parallel_loop(0, PCH, step=8, unroll=2)
        def _(e0):
            w16 = w_l[sm][pl.ds(e0, 16)]
            for l in range(8):
                wl = w16[l]
                for c8 in range(C // 16):
                    rows_l[sr][e0 + l, pl.ds(c8 * 16, 16)] = (
                        rows_l[sr][e0 + l, pl.ds(c8 * 16, 16)] * wl)

    def step(j, k, swait_pred, do_meta, do_gather):
        sr, sm = k % DRING, k % MRING
        srp, smp = (k + 2) % DRING, (k + 2) % MRING
        smi = (k + 4) % MRING

        if swait_pred is not None:
            @pl.when(swait_pred)
            def _():
                s_desc(srp, (k + 6) % MRING).wait()
        else:
            s_desc(srp, (k + 6) % MRING).wait()
        if do_meta:
            issue_meta(smi, j + 4)
        if do_gather:
            wait_meta(smp, j + 2)
            g_desc(srp, smp).start()
        g_desc(sr, sm).wait()
        scale(sr, sm)
        s_desc(sr, sm).start(add=True)

    for j0 in range(4):
        issue_meta(j0, j0)
    wait_meta(0, 0)
    g_desc(0, 0).start()
    wait_meta(1, 1)
    g_desc(1, 1).start()

    def super_chunk(t, carry):
        j = t * SUPER
        for k in range(SUPER):
            step(j + k, k, swait_pred=(j + k >= 2),
                 do_meta=True, do_gather=True)
        return carry

    lax.fori_loop(0, PNCH // SUPER, super_chunk, 0)
    for j in range(PNCH - NTAIL, PNCH):
        step(j, j % SUPER, swait_pred=None,
             do_meta=(j + 4 < PNCH), do_gather=(j + 2 < PNCH))
    s_desc(3, 3).wait()
    s_desc(0, 4).wait()
    plsc.subcore_barrier()
    for r in range(RPT // PCH):
        pltpu.sync_copy(acc_sh.at[pl.ds(sid * RPT + r * PCH, PCH)],
                        out_hbm.at[cid].at[pl.ds(sid * RPT + r * PCH, PCH)])


@functools.cache
def _sc_kernels():
    mesh = plsc.VectorSubcoreMesh(
        core_axis_name="c", subcore_axis_name="s",
        num_cores=NC, num_subcores=NS)
    sc_deg = pl.kernel(
        _sc_deg_body,
        out_type=jax.ShapeDtypeStruct((NC * NPAD,), jnp.float32),
        mesh=mesh,
        scratch_types=[
            pltpu.VMEM((EPT,), jnp.int32),
            pltpu.VMEM((EPT,), jnp.float32),
            pltpu.VMEM((DCH,), jnp.int32),
            pltpu.VMEM((DCH,), jnp.float32),
            pltpu.VMEM((RPT,), jnp.float32),
            pltpu.VMEM_SHARED((NPAD,), jnp.float32),
        ],
    )
    sc_prop = pl.kernel(
        _sc_prop_body,
        out_type=jax.ShapeDtypeStruct((NC, NPAD, C), jnp.float32),
        mesh=mesh,
        scratch_types=[
            tuple(pltpu.VMEM((PCH,), jnp.int32) for _ in range(MRING)),
            tuple(pltpu.VMEM((PCH,), jnp.int32) for _ in range(MRING)),
            tuple(pltpu.VMEM((96,), jnp.float32) for _ in range(MRING)),
            tuple(pltpu.VMEM((PCH, C), jnp.float32) for _ in range(DRING)),
            pltpu.VMEM_SHARED((NPAD, C), jnp.float32),
            tuple(pltpu.SemaphoreType.DMA for _ in range(MRING)),
            tuple(pltpu.SemaphoreType.DMA for _ in range(DRING)),
            tuple(pltpu.SemaphoreType.DMA for _ in range(DRING)),
        ],
    )
    return sc_deg, sc_prop


BS = 1000
GRID = N // BS


def _tc_pre_body(dp_ref, x_ref, dinv_ref, u0_ref):
    deg = dp_ref[0] + dp_ref[1]
    pos = deg > 0.0
    dinv = jnp.where(pos, lax.rsqrt(jnp.where(pos, deg, 1.0)), 0.0)
    dinv_ref[...] = dinv
    u0_ref[...] = dinv * x_ref[...]


_tc_pre = pl.pallas_call(
    _tc_pre_body,
    grid=(GRID,),
    in_specs=[
        pl.BlockSpec((NC, BS, 1), lambda i: (0, i, 0)),
        pl.BlockSpec((BS, C), lambda i: (i, 0)),
    ],
    out_specs=[
        pl.BlockSpec((BS, 1), lambda i: (i, 0)),
        pl.BlockSpec((BS, C), lambda i: (i, 0)),
    ],
    out_shape=[
        jax.ShapeDtypeStruct((N, 1), jnp.float32),
        jax.ShapeDtypeStruct((N, C), jnp.float32),
    ],
)


def _tc_mid_body(yp_ref, dinv_ref, x_ref, w0_ref, w1_ref, u1_ref, part_ref):
    y = yp_ref[0] + yp_ref[1]
    dinv = dinv_ref[...]
    tx1 = -dinv * y
    u1_ref[...] = dinv * tx1
    part_ref[...] = (
        jnp.dot(x_ref[...], w0_ref[...], preferred_element_type=jnp.float32)
        + jnp.dot(tx1, w1_ref[...], preferred_element_type=jnp.float32))


_tc_mid = pl.pallas_call(
    _tc_mid_body,
    grid=(GRID,),
    in_specs=[
        pl.BlockSpec((NC, BS, C), lambda i: (0, i, 0)),
        pl.BlockSpec((BS, 1), lambda i: (i, 0)),
        pl.BlockSpec((BS, C), lambda i: (i, 0)),
        pl.BlockSpec((C, C), lambda i: (0, 0)),
        pl.BlockSpec((C, C), lambda i: (0, 0)),
    ],
    out_specs=[
        pl.BlockSpec((BS, C), lambda i: (i, 0)),
        pl.BlockSpec((BS, C), lambda i: (i, 0)),
    ],
    out_shape=[
        jax.ShapeDtypeStruct((N, C), jnp.float32),
        jax.ShapeDtypeStruct((N, C), jnp.float32),
    ],
)


def _tc_post_body(yp_ref, dinv_ref, x_ref, part_ref, w2_ref, b_ref, o_ref):
    y = yp_ref[0] + yp_ref[1]
    tx2 = -2.0 * dinv_ref[...] * y - x_ref[...]
    o = (part_ref[...]
         + jnp.dot(tx2, w2_ref[...], preferred_element_type=jnp.float32)
         + b_ref[...])
    o_ref[...] = jnp.where(o >= 0.0, o, 0.01 * o)


_tc_post = pl.pallas_call(
    _tc_post_body,
    grid=(GRID,),
    in_specs=[
        pl.BlockSpec((NC, BS, C), lambda i: (0, i, 0)),
        pl.BlockSpec((BS, 1), lambda i: (i, 0)),
        pl.BlockSpec((BS, C), lambda i: (i, 0)),
        pl.BlockSpec((BS, C), lambda i: (i, 0)),
        pl.BlockSpec((C, C), lambda i: (0, 0)),
        pl.BlockSpec((1, C), lambda i: (0, 0)),
    ],
    out_specs=pl.BlockSpec((BS, C), lambda i: (i, 0)),
    out_shape=jax.ShapeDtypeStruct((N, C), jnp.float32),
)


def kernel(x, edgeIndex, edgeAttribute, W, b):
    row = edgeIndex[0]
    col = edgeIndex[1]
    _sc_deg, _sc_prop = _sc_kernels()

    deg_part = _sc_deg(row, edgeAttribute)
    dp = deg_part.reshape(NC, NPAD, 1)
    dinv, u0 = _tc_pre(dp, x)
    y1p = _sc_prop(u0, row, col, edgeAttribute)
    u1, part = _tc_mid(y1p, dinv, x, W[0], W[1])
    y2p = _sc_prop(u1, row, col, edgeAttribute)
    out = _tc_post(y2p, dinv, x, part, W[2], b.reshape(1, C))
    return out


# speedup vs baseline: 2.4666x; 1.0367x over previous
import functools

import jax
import jax.numpy as jnp
from jax import lax
from jax.experimental import pallas as pl
from jax.experimental.pallas import tpu as pltpu
from jax.experimental.pallas import tpu_sc as plsc

N = 10000
E = 320000
C = 128
NC = 2
NS = 16
NW = NC * NS
NPAD = 10240
EPT = E // NW
DCH = 80
DNCH = EPT // DCH
PCH = 80
PNCH = EPT // PCH
MRING = 8
DRING = 4
SUPER = 8
NTAIL = PNCH % SUPER
RPT = NPAD // NS

DRING5 = 5


def _sc_deg_body(row_hbm, ea_hbm, out_hbm, rbig_v, wbig_v, ridx_l, wsm_l,
                 zb_v, acc_sh, ssems):
    cid = lax.axis_index("c")
    sid = lax.axis_index("s")
    wid = cid * NS + sid
    pltpu.sync_copy(row_hbm.at[pl.ds(wid * EPT, EPT)], rbig_v)
    pltpu.sync_copy(ea_hbm.at[pl.ds(wid * EPT, EPT)], wbig_v)

    def zero_zb(i, carry):
        zb_v[pl.ds(i * 16, 16)] = jnp.zeros((16,), jnp.float32)
        return carry

    lax.fori_loop(0, RPT // 16, zero_zb, 0)
    pltpu.sync_copy(zb_v, acc_sh.at[pl.ds(sid * RPT, RPT)])
    plsc.subcore_barrier()

    def s_desc(s):
        return pltpu.make_async_copy(wsm_l[s], acc_sh.at[ridx_l[s]],
                                     ssems[s])

    def super_chunk(t, carry):
        for s in range(DRING5):
            i = t * DRING5 + s

            @pl.when(i >= DRING5)
            def _():
                s_desc(s).wait()

            def cp(g, c2, _s=s):
                ridx_l[_s][pl.ds(g * 16, 16)] = (
                    rbig_v[pl.ds(i * DCH + g * 16, 16)])
                wsm_l[_s][pl.ds(g * 16, 16)] = (
                    wbig_v[pl.ds(i * DCH + g * 16, 16)])
                return c2

            lax.fori_loop(0, DCH // 16, cp, 0)
            s_desc(s).start(add=True)
        return carry

    lax.fori_loop(0, DNCH // DRING5, super_chunk, 0)
    for s in range(DRING5):
        s_desc(s).wait()
    plsc.subcore_barrier()
    pltpu.sync_copy(acc_sh.at[pl.ds(sid * RPT, RPT)],
                    out_hbm.at[pl.ds(cid * NPAD + sid * RPT, RPT)])


def _sc_prop_body(u_hbm, row_hbm, col_hbm, ea_hbm, out_hbm,
                  ridx_l, cidx_l, w_l, rows_l, acc_sh,
                  msems, gsems, ssems):
    cid = lax.axis_index("c")
    sid = lax.axis_index("s")
    wid = cid * NS + sid
    base = wid * EPT

    def meta_descs(sm, j):
        off = base + j * PCH
        return (
            pltpu.make_async_copy(row_hbm.at[pl.ds(off, PCH)],
                                  ridx_l[sm], msems[sm]),
            pltpu.make_async_copy(col_hbm.at[pl.ds(off, PCH)],
                                  cidx_l[sm], msems[sm]),
            pltpu.make_async_copy(ea_hbm.at[pl.ds(off, PCH)],
                                  w_l[sm].at[pl.ds(0, PCH)], msems[sm]),
        )

    def issue_meta(sm, j):
        for d in meta_descs(sm, j):
            d.start()

    def wait_meta(sm, j):
        for d in meta_descs(sm, j):
            d.wait()

    def g_desc(sr, sm):
        return pltpu.make_async_copy(u_hbm.at[ridx_l[sm]], rows_l[sr],
                                     gsems[sr])

    def s_desc(sr, sm):
        return pltpu.make_async_copy(rows_l[sr], acc_sh.at[cidx_l[sm]],
                                     ssems[sr])

    def zero_rows(i, carry):
        for c8 in range(C // 16):
            rows_l[0][i, pl.ds(c8 * 16, 16)] = jnp.zeros((16,), jnp.float32)
        return carry

    lax.fori_loop(0, PCH, zero_rows, 0)
    zdescs = [
        pltpu.make_async_copy(
            rows_l[0], acc_sh.at[pl.ds(sid * RPT + r * PCH, PCH)], msems[r])
        for r in range(RPT // PCH)
    ]
    for d in zdescs:
        d.start()
    for d in zdescs:
        d.wait()
    plsc.subcore_barrier()

    def scale(sr, sm):
        @plsc.parallel_loop(0, PCH, step=8, unroll=2)
        def _(e0):
            w16 = w_l[sm][pl.ds(e0, 16)]
            for l in range(8):
                wl = w16[l]
                for c8 in range(C // 16):
                    rows_l[sr][e0 + l, pl.ds(c8 * 16, 16)] = (
                        rows_l[sr][e0 + l, pl.ds(c8 * 16, 16)] * wl)

    def step(j, k, swait_pred, do_meta, do_gather):
        sr, sm = k % DRING, k % MRING
        srp, smp = (k + 2) % DRING, (k + 2) % MRING
        smi = (k + 4) % MRING

        if swait_pred is not None:
            @pl.when(swait_pred)
            def _():
                s_desc(srp, (k + 6) % MRING).wait()
        else:
            s_desc(srp, (k + 6) % MRING).wait()
        if do_meta:
            issue_meta(smi, j + 4)
        if do_gather:
            wait_meta(smp, j + 2)
            g_desc(srp, smp).start()
        g_desc(sr, sm).wait()
        scale(sr, sm)
        s_desc(sr, sm).start(add=True)

    for j0 in range(4):
        issue_meta(j0, j0)
    wait_meta(0, 0)
    g_desc(0, 0).start()
    wait_meta(1, 1)
    g_desc(1, 1).start()

    def super_chunk(t, carry):
        j = t * SUPER
        for k in range(SUPER):
            step(j + k, k, swait_pred=(j + k >= 2),
                 do_meta=True, do_gather=True)
        return carry

    lax.fori_loop(0, PNCH // SUPER, super_chunk, 0)
    for j in range(PNCH - NTAIL, PNCH):
        step(j, j % SUPER, swait_pred=None,
             do_meta=(j + 4 < PNCH), do_gather=(j + 2 < PNCH))
    s_desc(3, 3).wait()
    s_desc(0, 4).wait()
    plsc.subcore_barrier()
    odescs = [
        pltpu.make_async_copy(
            acc_sh.at[pl.ds(sid * RPT + r * PCH, PCH)],
            out_hbm.at[cid].at[pl.ds(sid * RPT + r * PCH, PCH)], msems[r])
        for r in range(RPT // PCH)
    ]
    for d in odescs:
        d.start()
    for d in odescs:
        d.wait()


@functools.cache
def _sc_kernels():
    mesh = plsc.VectorSubcoreMesh(
        core_axis_name="c", subcore_axis_name="s",
        num_cores=NC, num_subcores=NS)
    sc_deg = pl.kernel(
        _sc_deg_body,
        out_type=jax.ShapeDtypeStruct((NC * NPAD,), jnp.float32),
        mesh=mesh,
        scratch_types=[
            pltpu.VMEM((EPT,), jnp.int32),
            pltpu.VMEM((EPT,), jnp.float32),
            tuple(pltpu.VMEM((DCH,), jnp.int32) for _ in range(DRING5)),
            tuple(pltpu.VMEM((DCH,), jnp.float32) for _ in range(DRING5)),
            pltpu.VMEM((RPT,), jnp.float32),
            pltpu.VMEM_SHARED((NPAD,), jnp.float32),
            tuple(pltpu.SemaphoreType.DMA for _ in range(DRING5)),
        ],
    )
    sc_prop = pl.kernel(
        _sc_prop_body,
        out_type=jax.ShapeDtypeStruct((NC, NPAD, C), jnp.float32),
        mesh=mesh,
        scratch_types=[
            tuple(pltpu.VMEM((PCH,), jnp.int32) for _ in range(MRING)),
            tuple(pltpu.VMEM((PCH,), jnp.int32) for _ in range(MRING)),
            tuple(pltpu.VMEM((96,), jnp.float32) for _ in range(MRING)),
            tuple(pltpu.VMEM((PCH, C), jnp.float32) for _ in range(DRING)),
            pltpu.VMEM_SHARED((NPAD, C), jnp.float32),
            tuple(pltpu.SemaphoreType.DMA for _ in range(MRING)),
            tuple(pltpu.SemaphoreType.DMA for _ in range(DRING)),
            tuple(pltpu.SemaphoreType.DMA for _ in range(DRING)),
        ],
    )
    return sc_deg, sc_prop


BS = 1000
GRID = N // BS


def _tc_pre_body(dp_ref, x_ref, dinv_ref, u0_ref):
    deg = dp_ref[0] + dp_ref[1]
    pos = deg > 0.0
    dinv = jnp.where(pos, lax.rsqrt(jnp.where(pos, deg, 1.0)), 0.0)
    dinv_ref[...] = dinv
    u0_ref[...] = dinv * x_ref[...]


_tc_pre = pl.pallas_call(
    _tc_pre_body,
    grid=(GRID,),
    in_specs=[
        pl.BlockSpec((NC, BS, 1), lambda i: (0, i, 0)),
        pl.BlockSpec((BS, C), lambda i: (i, 0)),
    ],
    out_specs=[
        pl.BlockSpec((BS, 1), lambda i: (i, 0)),
        pl.BlockSpec((BS, C), lambda i: (i, 0)),
    ],
    out_shape=[
        jax.ShapeDtypeStruct((N, 1), jnp.float32),
        jax.ShapeDtypeStruct((N, C), jnp.float32),
    ],
)


def _tc_mid_body(yp_ref, dinv_ref, x_ref, w0_ref, w1_ref, u1_ref, part_ref):
    y = yp_ref[0] + yp_ref[1]
    dinv = dinv_ref[...]
    tx1 = -dinv * y
    u1_ref[...] = dinv * tx1
    part_ref[...] = (
        jnp.dot(x_ref[...], w0_ref[...], preferred_element_type=jnp.float32)
        + jnp.dot(tx1, w1_ref[...], preferred_element_type=jnp.float32))


_tc_mid = pl.pallas_call(
    _tc_mid_body,
    grid=(GRID,),
    in_specs=[
        pl.BlockSpec((NC, BS, C), lambda i: (0, i, 0)),
        pl.BlockSpec((BS, 1), lambda i: (i, 0)),
        pl.BlockSpec((BS, C), lambda i: (i, 0)),
        pl.BlockSpec((C, C), lambda i: (0, 0)),
        pl.BlockSpec((C, C), lambda i: (0, 0)),
    ],
    out_specs=[
        pl.BlockSpec((BS, C), lambda i: (i, 0)),
        pl.BlockSpec((BS, C), lambda i: (i, 0)),
    ],
    out_shape=[
        jax.ShapeDtypeStruct((N, C), jnp.float32),
        jax.ShapeDtypeStruct((N, C), jnp.float32),
    ],
)


def _tc_post_body(yp_ref, dinv_ref, x_ref, part_ref, w2_ref, b_ref, o_ref):
    y = yp_ref[0] + yp_ref[1]
    tx2 = -2.0 * dinv_ref[...] * y - x_ref[...]
    o = (part_ref[...]
         + jnp.dot(tx2, w2_ref[...], preferred_element_type=jnp.float32)
         + b_ref[...])
    o_ref[...] = jnp.where(o >= 0.0, o, 0.01 * o)


_tc_post = pl.pallas_call(
    _tc_post_body,
    grid=(GRID,),
    in_specs=[
        pl.BlockSpec((NC, BS, C), lambda i: (0, i, 0)),
        pl.BlockSpec((BS, 1), lambda i: (i, 0)),
        pl.BlockSpec((BS, C), lambda i: (i, 0)),
        pl.BlockSpec((BS, C), lambda i: (i, 0)),
        pl.BlockSpec((C, C), lambda i: (0, 0)),
        pl.BlockSpec((1, C), lambda i: (0, 0)),
    ],
    out_specs=pl.BlockSpec((BS, C), lambda i: (i, 0)),
    out_shape=jax.ShapeDtypeStruct((N, C), jnp.float32),
)


def kernel(x, edgeIndex, edgeAttribute, W, b):
    row = edgeIndex[0]
    col = edgeIndex[1]
    _sc_deg, _sc_prop = _sc_kernels()

    deg_part = _sc_deg(row, edgeAttribute)
    dp = deg_part.reshape(NC, NPAD, 1)
    dinv, u0 = _tc_pre(dp, x)
    y1p = _sc_prop(u0, row, col, edgeAttribute)
    u1, part = _tc_mid(y1p, dinv, x, W[0], W[1])
    y2p = _sc_prop(u1, row, col, edgeAttribute)
    out = _tc_post(y2p, dinv, x, part, W[2], b.reshape(1, C))
    return out
